# Initial kernel scaffold; baseline (speedup 1.0000x reference)
import jax
import jax.numpy as jnp
from jax.experimental import pallas as pl


def kernel(x, edge_index, W1, att_src1, att_dst1, b1, W2, att_src2, att_dst2, b2):
    raise NotImplementedError("write your pallas kernel here")


# speedup vs baseline: 33.9718x; 33.9718x over previous
import functools

import jax
import jax.numpy as jnp
from jax import lax
from jax.experimental import pallas as pl
from jax.experimental.pallas import tpu as pltpu
from jax.experimental.pallas import tpu_sc as plsc

N = 10000
E = 320000
DIN = 128
HEADS = 8
HID = 16
D1 = HEADS * HID
DOUT = 40

NC = 2
NS = 16
LANES = 16
NW = NC * NS
EPT = E // NW
K = 80
NCHUNK = EPT // K
NPAD = 10240
RPT = NPAD // NS
RCH = 64

ROW1 = D1 + 16
CH2 = 48
ROW2 = CH2 + 16

_f32 = jnp.float32
_i32 = jnp.int32


def _dense1_body(x_ref, w1_ref, as_ref, ad_ref, h_ref, ap_ref, bp_ref, g_ref):
    h = jnp.dot(x_ref[...], w1_ref[...], preferred_element_type=_f32)
    h_ref[...] = h
    ap = jnp.dot(h, as_ref[...], preferred_element_type=_f32)
    bp = jnp.dot(h, ad_ref[...], preferred_element_type=_f32)
    ap_ref[...] = ap
    bp_ref[...] = bp
    gm = jnp.max(ap, axis=0) + jnp.max(bp, axis=0)
    g = jnp.maximum(gm, 0.2 * gm)
    g_ref[...] = jnp.broadcast_to(g[None, :], (8, 16))


_dense1 = pl.pallas_call(
    _dense1_body,
    out_shape=[
        jax.ShapeDtypeStruct((N, D1), _f32),
        jax.ShapeDtypeStruct((N, 16), _f32),
        jax.ShapeDtypeStruct((N, 16), _f32),
        jax.ShapeDtypeStruct((8, 16), _f32),
    ],
)


def _dense2_body(acc_ref, b1_ref, w2p_ref, a2s_ref, a2d_ref, r_ref,
                 h2_ref, ap2_ref, bp2_ref, g2_ref):
    num = acc_ref[0:N, 0:D1] + acc_ref[NPAD:NPAD + N, 0:D1]
    den8 = acc_ref[0:N, D1:ROW1] + acc_ref[NPAD:NPAD + N, D1:ROW1]
    den128 = jnp.dot(den8, r_ref[...], preferred_element_type=_f32)
    z = jnp.maximum(num / (den128 + 1e-16) + b1_ref[...], 0.0)
    h2 = jnp.dot(z, w2p_ref[...], preferred_element_type=_f32)
    h2_ref[...] = h2
    ap2 = jnp.dot(h2, a2s_ref[...], preferred_element_type=_f32)
    bp2 = jnp.dot(h2, a2d_ref[...], preferred_element_type=_f32)
    ap2_ref[...] = ap2
    bp2_ref[...] = bp2
    gm = jnp.max(ap2, axis=0) + jnp.max(bp2, axis=0)
    g2 = jnp.maximum(gm, 0.2 * gm)
    g2_ref[...] = jnp.broadcast_to(g2[None, :], (8, 16))


_dense2 = pl.pallas_call(
    _dense2_body,
    out_shape=[
        jax.ShapeDtypeStruct((N, CH2), _f32),
        jax.ShapeDtypeStruct((N, 16), _f32),
        jax.ShapeDtypeStruct((N, 16), _f32),
        jax.ShapeDtypeStruct((8, 16), _f32),
    ],
)


def _dense3_body(acc_ref, b2_ref, out_ref):
    num = acc_ref[0:N, 0:DOUT] + acc_ref[NPAD:NPAD + N, 0:DOUT]
    den = (acc_ref[0:N, CH2:CH2 + 1]
           + acc_ref[NPAD:NPAD + N, CH2:CH2 + 1])
    o = num / (den + 1e-16) + b2_ref[...]
    m = jnp.max(o, axis=1, keepdims=True)
    e = jnp.exp(o - m)
    out_ref[...] = o - m - jnp.log(jnp.sum(e, axis=1, keepdims=True))


_dense3 = pl.pallas_call(
    _dense3_body,
    out_shape=jax.ShapeDtypeStruct((N, DOUT), _f32),
)


def _edge_body(nch, row_w, n_hvec, heads, h_hbm, ap_hbm, bp_hbm, g_hbm,
               src_hbm, dst_hbm, out_hbm, acc, srcv, dstv, rowsv, av, bv,
               outv, gv, zbuf, sem0, sem1, sem2):
    c = lax.axis_index("c")
    s = lax.axis_index("s")
    wid = s * NC + c
    chw = n_hvec * LANES

    pltpu.sync_copy(g_hbm.at[0], gv)

    def zrow(r, carry):
        for j in range(row_w // LANES):
            zbuf[r, pl.ds(j * LANES, LANES)] = jnp.zeros((LANES,), _f32)
        return carry

    lax.fori_loop(0, RCH, zrow, 0)
    for j in range(RPT // RCH):
        pltpu.sync_copy(zbuf, acc.at[pl.ds(s * RPT + j * RCH, RCH)])
    plsc.subcore_barrier()

    gvec = gv[...]
    ebase = wid * EPT

    def chunk(i, carry):
        base = ebase + i * K
        pltpu.sync_copy(src_hbm.at[pl.ds(base, K)], srcv)
        pltpu.sync_copy(dst_hbm.at[pl.ds(base, K)], dstv)
        d0 = pltpu.async_copy(h_hbm.at[srcv], rowsv, sem0)
        d1 = pltpu.async_copy(ap_hbm.at[srcv], av, sem1)
        d2 = pltpu.async_copy(bp_hbm.at[dstv], bv, sem2)
        d0.wait()
        d1.wait()
        d2.wait()

        def edge(k, icarry):
            t = av[k, :] + bv[k, :]
            w = jnp.exp(jnp.maximum(t, 0.2 * t) - gvec)
            outv[k, pl.ds(chw, LANES)] = w
            for h in range(n_hvec):
                wb = lax.gather(
                    w, jnp.full((LANES, 1), h % heads, _i32),
                    lax.GatherDimensionNumbers(
                        offset_dims=(), collapsed_slice_dims=(0,),
                        start_index_map=(0,)),
                    (1,), mode=lax.GatherScatterMode.PROMISE_IN_BOUNDS)
                outv[k, pl.ds(h * LANES, LANES)] = (
                    wb * rowsv[k, pl.ds(h * LANES, LANES)])
            return icarry

        lax.fori_loop(0, K, edge, 0)
        pltpu.sync_copy(outv, acc.at[dstv], add=True)
        return carry

    lax.fori_loop(0, nch, chunk, 0)
    plsc.subcore_barrier()

    for j in range(RPT // RCH):
        start = s * RPT + j * RCH
        pltpu.sync_copy(acc.at[pl.ds(start, RCH)], zbuf)
        pltpu.sync_copy(zbuf, out_hbm.at[pl.ds(c * NPAD + start, RCH)])


def _make_edge_kernel(row_w, n_hvec, heads):
    mesh = plsc.VectorSubcoreMesh(
        core_axis_name="c", subcore_axis_name="s",
        num_cores=NC, num_subcores=NS)
    chw = n_hvec * LANES
    return pl.kernel(
        functools.partial(_edge_body, NCHUNK, row_w, n_hvec, heads),
        out_type=jax.ShapeDtypeStruct((2 * NPAD, row_w), _f32),
        mesh=mesh,
        compiler_params=pltpu.CompilerParams(use_tc_tiling_on_sc=False),
        scratch_types=[
            pltpu.MemorySpace.VMEM_SHARED((NPAD, row_w), _f32),
            pltpu.VMEM((K,), _i32),
            pltpu.VMEM((K,), _i32),
            pltpu.VMEM((K, chw), _f32),
            pltpu.VMEM((K, 16), _f32),
            pltpu.VMEM((K, 16), _f32),
            pltpu.VMEM((K, row_w), _f32),
            pltpu.VMEM((LANES,), _f32),
            pltpu.VMEM((RCH, row_w), _f32),
            pltpu.SemaphoreType.DMA,
            pltpu.SemaphoreType.DMA,
            pltpu.SemaphoreType.DMA,
        ],
    )


_edge1 = _make_edge_kernel(ROW1, D1 // LANES, HEADS)
_edge2 = _make_edge_kernel(ROW2, CH2 // LANES, 1)


def _head_selector(att, heads, hid):
    a = (att.reshape(heads, hid, 1) *
         jnp.eye(heads, dtype=_f32)[:, None, :]).reshape(heads * hid, heads)
    return jnp.pad(a, ((0, 0), (0, 16 - heads)))


def kernel(x, edge_index, W1, att_src1, att_dst1, b1, W2, att_src2,
           att_dst2, b2):
    src = edge_index[0].astype(_i32)
    dst = edge_index[1].astype(_i32)

    a1s = _head_selector(att_src1, HEADS, HID)
    a1d = _head_selector(att_dst1, HEADS, HID)
    w2p = jnp.pad(W2, ((0, 0), (0, CH2 - DOUT)))
    a2s = jnp.pad(att_src2.reshape(DOUT, 1), ((0, CH2 - DOUT), (0, 15)))
    a2d = jnp.pad(att_dst2.reshape(DOUT, 1), ((0, CH2 - DOUT), (0, 15)))
    rsel = (jnp.arange(D1)[None, :] // HID ==
            jnp.arange(16)[:, None]).astype(_f32)
    b1r = b1.reshape(1, D1)
    b2r = b2.reshape(1, DOUT)

    h1, ap1, bp1, g1 = _dense1(x, W1, a1s, a1d)
    acc1 = _edge1(h1, ap1, bp1, g1, src, dst)
    h2, ap2, bp2, g2 = _dense2(acc1, b1r, w2p, a2s, a2d, rsel)
    acc2 = _edge2(h2, ap2, bp2, g2, src, dst)
    return _dense3(acc2, b2r)


# speedup vs baseline: 43.9385x; 1.2934x over previous
import functools

import jax
import jax.numpy as jnp
from jax import lax
from jax.experimental import pallas as pl
from jax.experimental.pallas import tpu as pltpu
from jax.experimental.pallas import tpu_sc as plsc

N = 10000
E = 320000
DIN = 128
HEADS = 8
HID = 16
D1 = HEADS * HID
DOUT = 40

NC = 2
NS = 16
LANES = 16
NW = NC * NS
EPT = E // NW
K = 40
NCHUNK = EPT // K
G = 10
NGROUP = NCHUNK // G
NPAD = 10240
RPT = NPAD // NS
RCH = 64

ROW1 = D1 + 16
CH2 = 48
ROW2 = CH2 + 16

_f32 = jnp.float32
_i32 = jnp.int32


def _dense1_body(x_ref, w1_ref, as_ref, ad_ref, h_ref, ap_ref, bp_ref, g_ref):
    h = jnp.dot(x_ref[...], w1_ref[...], preferred_element_type=_f32)
    h_ref[...] = h
    ap = jnp.dot(h, as_ref[...], preferred_element_type=_f32)
    bp = jnp.dot(h, ad_ref[...], preferred_element_type=_f32)
    ap_ref[...] = ap
    bp_ref[...] = bp
    gm = jnp.max(ap, axis=0) + jnp.max(bp, axis=0)
    g = jnp.maximum(gm, 0.2 * gm)
    g_ref[...] = jnp.broadcast_to(g[None, :], (8, 16))


_dense1 = pl.pallas_call(
    _dense1_body,
    out_shape=[
        jax.ShapeDtypeStruct((N, D1), _f32),
        jax.ShapeDtypeStruct((N, 16), _f32),
        jax.ShapeDtypeStruct((N, 16), _f32),
        jax.ShapeDtypeStruct((8, 16), _f32),
    ],
)


def _dense2_body(acc_ref, b1_ref, w2p_ref, a2s_ref, a2d_ref, r_ref,
                 h2_ref, ap2_ref, bp2_ref, g2_ref):
    num = acc_ref[0:N, 0:D1] + acc_ref[NPAD:NPAD + N, 0:D1]
    den8 = acc_ref[0:N, D1:ROW1] + acc_ref[NPAD:NPAD + N, D1:ROW1]
    den128 = jnp.dot(den8, r_ref[...], preferred_element_type=_f32)
    z = jnp.maximum(num / (den128 + 1e-16) + b1_ref[...], 0.0)
    h2 = jnp.dot(z, w2p_ref[...], preferred_element_type=_f32)
    h2_ref[...] = h2
    ap2 = jnp.dot(h2, a2s_ref[...], preferred_element_type=_f32)
    bp2 = jnp.dot(h2, a2d_ref[...], preferred_element_type=_f32)
    ap2_ref[...] = ap2
    bp2_ref[...] = bp2
    gm = jnp.max(ap2, axis=0) + jnp.max(bp2, axis=0)
    g2 = jnp.maximum(gm, 0.2 * gm)
    g2_ref[...] = jnp.broadcast_to(g2[None, :], (8, 16))


_dense2 = pl.pallas_call(
    _dense2_body,
    out_shape=[
        jax.ShapeDtypeStruct((N, CH2), _f32),
        jax.ShapeDtypeStruct((N, 16), _f32),
        jax.ShapeDtypeStruct((N, 16), _f32),
        jax.ShapeDtypeStruct((8, 16), _f32),
    ],
)


def _dense3_body(acc_ref, b2_ref, out_ref):
    num = acc_ref[0:N, 0:DOUT] + acc_ref[NPAD:NPAD + N, 0:DOUT]
    den = (acc_ref[0:N, CH2:CH2 + 1]
           + acc_ref[NPAD:NPAD + N, CH2:CH2 + 1])
    o = num / (den + 1e-16) + b2_ref[...]
    m = jnp.max(o, axis=1, keepdims=True)
    e = jnp.exp(o - m)
    out_ref[...] = o - m - jnp.log(jnp.sum(e, axis=1, keepdims=True))


_dense3 = pl.pallas_call(
    _dense3_body,
    out_shape=jax.ShapeDtypeStruct((N, DOUT), _f32),
)


def _edge_body(row_w, n_hvec, heads, h_hbm, ap_hbm, bp_hbm, g_hbm,
               src_hbm, dst_hbm, out_hbm, acc, srcg, dstg, rows0, rows1,
               av0, av1, bv0, bv1, outv, gv, zbuf,
               sr0, sr1, sa0, sa1, sb0, sb1):
    c = lax.axis_index("c")
    s = lax.axis_index("s")
    wid = s * NC + c
    chw = n_hvec * LANES
    rows_ = (rows0, rows1)
    av_ = (av0, av1)
    bv_ = (bv0, bv1)
    sems = ((sr0, sa0, sb0), (sr1, sa1, sb1))

    pltpu.sync_copy(g_hbm.at[0], gv)

    def zrow(r, carry):
        for j in range(row_w // LANES):
            zbuf[r, pl.ds(j * LANES, LANES)] = jnp.zeros((LANES,), _f32)
        return carry

    lax.fori_loop(0, RCH, zrow, 0)
    for j in range(RPT // RCH):
        pltpu.sync_copy(zbuf, acc.at[pl.ds(s * RPT + j * RCH, RCH)])
    plsc.subcore_barrier()

    gvec = gv[...]
    wbase = wid * NCHUNK

    def fire(j, b):
        d0 = pltpu.async_copy(h_hbm.at[srcg.at[j]], rows_[b], sems[b][0])
        d1 = pltpu.async_copy(ap_hbm.at[srcg.at[j]], av_[b], sems[b][1])
        d2 = pltpu.async_copy(bp_hbm.at[dstg.at[j]], bv_[b], sems[b][2])
        return (d0, d1, d2)

    def compute(b):
        rv, avb, bvb = rows_[b], av_[b], bv_[b]

        def edge(k, icarry):
            t = avb[k, :] + bvb[k, :]
            w = jnp.exp(jnp.maximum(t, 0.2 * t) - gvec)
            outv[k, pl.ds(chw, LANES)] = w
            for h in range(n_hvec):
                wb = lax.gather(
                    w, jnp.full((LANES, 1), h % heads, _i32),
                    lax.GatherDimensionNumbers(
                        offset_dims=(), collapsed_slice_dims=(0,),
                        start_index_map=(0,)),
                    (1,), mode=lax.GatherScatterMode.PROMISE_IN_BOUNDS)
                outv[k, pl.ds(h * LANES, LANES)] = (
                    wb * rv[k, pl.ds(h * LANES, LANES)])
            return icarry

        lax.fori_loop(0, K, edge, 0)

    def group(g, carry):
        pltpu.sync_copy(src_hbm.at[pl.ds(wbase + g * G, G)], srcg)
        pltpu.sync_copy(dst_hbm.at[pl.ds(wbase + g * G, G)], dstg)
        pend = fire(0, 0)
        for j in range(G):
            b = j % 2
            nxt = fire(j + 1, 1 - b) if j + 1 < G else None
            for d in pend:
                d.wait()
            compute(b)
            pltpu.sync_copy(outv, acc.at[dstg.at[j]], add=True)
            pend = nxt
        return carry

    lax.fori_loop(0, NGROUP, group, 0)
    plsc.subcore_barrier()

    for j in range(RPT // RCH):
        start = s * RPT + j * RCH
        pltpu.sync_copy(acc.at[pl.ds(start, RCH)], zbuf)
        pltpu.sync_copy(zbuf, out_hbm.at[pl.ds(c * NPAD + start, RCH)])


def _make_edge_kernel(row_w, n_hvec, heads):
    mesh = plsc.VectorSubcoreMesh(
        core_axis_name="c", subcore_axis_name="s",
        num_cores=NC, num_subcores=NS)
    chw = n_hvec * LANES
    dma = pltpu.SemaphoreType.DMA
    return pl.kernel(
        functools.partial(_edge_body, row_w, n_hvec, heads),
        out_type=jax.ShapeDtypeStruct((2 * NPAD, row_w), _f32),
        mesh=mesh,
        compiler_params=pltpu.CompilerParams(use_tc_tiling_on_sc=False),
        scratch_types=[
            pltpu.MemorySpace.VMEM_SHARED((NPAD, row_w), _f32),
            pltpu.VMEM((G, K), _i32),
            pltpu.VMEM((G, K), _i32),
            pltpu.VMEM((K, chw), _f32),
            pltpu.VMEM((K, chw), _f32),
            pltpu.VMEM((K, 16), _f32),
            pltpu.VMEM((K, 16), _f32),
            pltpu.VMEM((K, 16), _f32),
            pltpu.VMEM((K, 16), _f32),
            pltpu.VMEM((K, row_w), _f32),
            pltpu.VMEM((LANES,), _f32),
            pltpu.VMEM((RCH, row_w), _f32),
            dma, dma, dma, dma, dma, dma,
        ],
    )


_edge1 = _make_edge_kernel(ROW1, D1 // LANES, HEADS)
_edge2 = _make_edge_kernel(ROW2, CH2 // LANES, 1)


def _head_selector(att, heads, hid):
    a = (att.reshape(heads, hid, 1) *
         jnp.eye(heads, dtype=_f32)[:, None, :]).reshape(heads * hid, heads)
    return jnp.pad(a, ((0, 0), (0, 16 - heads)))


def kernel(x, edge_index, W1, att_src1, att_dst1, b1, W2, att_src2,
           att_dst2, b2):
    src = edge_index[0].astype(_i32).reshape(NW * NCHUNK, K)
    dst = edge_index[1].astype(_i32).reshape(NW * NCHUNK, K)

    a1s = _head_selector(att_src1, HEADS, HID)
    a1d = _head_selector(att_dst1, HEADS, HID)
    w2p = jnp.pad(W2, ((0, 0), (0, CH2 - DOUT)))
    a2s = jnp.pad(att_src2.reshape(DOUT, 1), ((0, CH2 - DOUT), (0, 15)))
    a2d = jnp.pad(att_dst2.reshape(DOUT, 1), ((0, CH2 - DOUT), (0, 15)))
    rsel = (jnp.arange(D1)[None, :] // HID ==
            jnp.arange(16)[:, None]).astype(_f32)
    b1r = b1.reshape(1, D1)
    b2r = b2.reshape(1, DOUT)

    h1, ap1, bp1, g1 = _dense1(x, W1, a1s, a1d)
    acc1 = _edge1(h1, ap1, bp1, g1, src, dst)
    h2, ap2, bp2, g2 = _dense2(acc1, b1r, w2p, a2s, a2d, rsel)
    acc2 = _edge2(h2, ap2, bp2, g2, src, dst)
    return _dense3(acc2, b2r)


# speedup vs baseline: 50.4991x; 1.1493x over previous
import functools

import jax
import jax.numpy as jnp
from jax import lax
from jax.experimental import pallas as pl
from jax.experimental.pallas import tpu as pltpu
from jax.experimental.pallas import tpu_sc as plsc

N = 10000
E = 320000
DIN = 128
HEADS = 8
HID = 16
D1 = HEADS * HID
DOUT = 40

NC = 2
NS = 16
LANES = 16
NW = NC * NS
EPT = E // NW
K = 40
NCHUNK = EPT // K
G = 25
NGROUP = NCHUNK // G
NPAD = 10240
RPT = NPAD // NS
RCH = 64

ROW1 = D1 + 16
CH2 = 48
ROW2 = CH2 + 16

_f32 = jnp.float32
_i32 = jnp.int32


def _dense1_body(x_ref, w1_ref, as_ref, ad_ref, h_ref, ap_ref, bp_ref, g_ref):
    h = jnp.dot(x_ref[...], w1_ref[...], preferred_element_type=_f32)
    h_ref[...] = h
    ap = jnp.dot(h, as_ref[...], preferred_element_type=_f32)
    bp = jnp.dot(h, ad_ref[...], preferred_element_type=_f32)
    ap_ref[...] = ap
    bp_ref[...] = bp
    gm = jnp.max(ap, axis=0) + jnp.max(bp, axis=0)
    g = jnp.maximum(gm, 0.2 * gm)
    g_ref[...] = jnp.broadcast_to(g[None, :], (8, 16))


_dense1 = pl.pallas_call(
    _dense1_body,
    out_shape=[
        jax.ShapeDtypeStruct((N, D1), _f32),
        jax.ShapeDtypeStruct((N, 16), _f32),
        jax.ShapeDtypeStruct((N, 16), _f32),
        jax.ShapeDtypeStruct((8, 16), _f32),
    ],
)


def _dense2_body(acc_ref, b1_ref, w2p_ref, a2s_ref, a2d_ref, r_ref,
                 h2_ref, ap2_ref, bp2_ref, g2_ref):
    num = acc_ref[0:N, 0:D1] + acc_ref[NPAD:NPAD + N, 0:D1]
    den8 = acc_ref[0:N, D1:ROW1] + acc_ref[NPAD:NPAD + N, D1:ROW1]
    den128 = jnp.dot(den8, r_ref[...], preferred_element_type=_f32)
    z = jnp.maximum(num / (den128 + 1e-16) + b1_ref[...], 0.0)
    h2 = jnp.dot(z, w2p_ref[...], preferred_element_type=_f32)
    h2_ref[...] = h2
    ap2 = jnp.dot(h2, a2s_ref[...], preferred_element_type=_f32)
    bp2 = jnp.dot(h2, a2d_ref[...], preferred_element_type=_f32)
    ap2_ref[...] = ap2
    bp2_ref[...] = bp2
    gm = jnp.max(ap2, axis=0) + jnp.max(bp2, axis=0)
    g2 = jnp.maximum(gm, 0.2 * gm)
    g2_ref[...] = jnp.broadcast_to(g2[None, :], (8, 16))


_dense2 = pl.pallas_call(
    _dense2_body,
    out_shape=[
        jax.ShapeDtypeStruct((N, CH2), _f32),
        jax.ShapeDtypeStruct((N, 16), _f32),
        jax.ShapeDtypeStruct((N, 16), _f32),
        jax.ShapeDtypeStruct((8, 16), _f32),
    ],
)


def _dense3_body(acc_ref, b2_ref, out_ref):
    num = acc_ref[0:N, 0:DOUT] + acc_ref[NPAD:NPAD + N, 0:DOUT]
    den = (acc_ref[0:N, CH2:CH2 + 1]
           + acc_ref[NPAD:NPAD + N, CH2:CH2 + 1])
    o = num / (den + 1e-16) + b2_ref[...]
    m = jnp.max(o, axis=1, keepdims=True)
    e = jnp.exp(o - m)
    out_ref[...] = o - m - jnp.log(jnp.sum(e, axis=1, keepdims=True))


_dense3 = pl.pallas_call(
    _dense3_body,
    out_shape=jax.ShapeDtypeStruct((N, DOUT), _f32),
)


def _edge_body(row_w, n_hvec, heads, h_hbm, ap_hbm, bp_hbm, g_hbm,
               src_hbm, dst_hbm, out_hbm, acc, srcg, dstg, rows0, rows1,
               av0, av1, bv0, bv1, out0, out1, gv, zbuf,
               sr0, sr1, sa0, sa1, sb0, sb1, sc0, sc1):
    c = lax.axis_index("c")
    s = lax.axis_index("s")
    wid = s * NC + c
    chw = n_hvec * LANES
    rows_ = (rows0, rows1)
    av_ = (av0, av1)
    bv_ = (bv0, bv1)
    out_ = (out0, out1)
    scsem = (sc0, sc1)
    sems = ((sr0, sa0, sb0), (sr1, sa1, sb1))

    pltpu.sync_copy(g_hbm.at[0], gv)

    def zrow(r, carry):
        for j in range(row_w // LANES):
            zbuf[r, pl.ds(j * LANES, LANES)] = jnp.zeros((LANES,), _f32)
        return carry

    lax.fori_loop(0, RCH, zrow, 0)
    for j in range(RPT // RCH):
        pltpu.sync_copy(zbuf, acc.at[pl.ds(s * RPT + j * RCH, RCH)])
    plsc.subcore_barrier()

    gvec = gv[...]
    widx = (lax.iota(_i32, LANES) % heads)[:, None]
    dnums = lax.GatherDimensionNumbers(
        offset_dims=(), collapsed_slice_dims=(0,), start_index_map=(0,))
    wbase = wid * NCHUNK

    def fire(j, b):
        d0 = pltpu.async_copy(h_hbm.at[srcg.at[j]], rows_[b], sems[b][0])
        d1 = pltpu.async_copy(ap_hbm.at[srcg.at[j]], av_[b], sems[b][1])
        d2 = pltpu.async_copy(bp_hbm.at[dstg.at[j]], bv_[b], sems[b][2])
        return (d0, d1, d2)

    def compute(b):
        rv, avb, bvb, ov = rows_[b], av_[b], bv_[b], out_[b]

        def edge2(i, icarry):
            for u in range(2):
                k = i * 2 + u
                t = avb[k, :] + bvb[k, :]
                w = jnp.exp(jnp.maximum(t, 0.2 * t) - gvec)
                ov[k, pl.ds(chw, LANES)] = w
                wp = lax.gather(
                    w, widx, dnums, (1,),
                    mode=lax.GatherScatterMode.PROMISE_IN_BOUNDS)
                for h in range(n_hvec):
                    ov[k, pl.ds(h * LANES, LANES)] = (
                        wp * rv[k, pl.ds(h * LANES, LANES)])
            return icarry

        lax.fori_loop(0, K // 2, edge2, 0)

    def group(g, carry):
        pltpu.sync_copy(src_hbm.at[pl.ds(wbase + g * G, G)], srcg)
        pltpu.sync_copy(dst_hbm.at[pl.ds(wbase + g * G, G)], dstg)
        pend = fire(0, 0)
        pend_sc = [None, None]
        for j in range(G):
            b = j % 2
            nxt = fire(j + 1, 1 - b) if j + 1 < G else None
            for d in pend:
                d.wait()
            if pend_sc[b] is not None:
                pend_sc[b].wait()
            compute(b)
            pend_sc[b] = pltpu.async_copy(
                out_[b], acc.at[dstg.at[j]], scsem[b], add=True)
            pend = nxt
        for d in pend_sc:
            if d is not None:
                d.wait()
        return carry

    lax.fori_loop(0, NGROUP, group, 0)
    plsc.subcore_barrier()

    for j in range(RPT // RCH):
        start = s * RPT + j * RCH
        pltpu.sync_copy(acc.at[pl.ds(start, RCH)], zbuf)
        pltpu.sync_copy(zbuf, out_hbm.at[pl.ds(c * NPAD + start, RCH)])


def _make_edge_kernel(row_w, n_hvec, heads):
    mesh = plsc.VectorSubcoreMesh(
        core_axis_name="c", subcore_axis_name="s",
        num_cores=NC, num_subcores=NS)
    chw = n_hvec * LANES
    dma = pltpu.SemaphoreType.DMA
    return pl.kernel(
        functools.partial(_edge_body, row_w, n_hvec, heads),
        out_type=jax.ShapeDtypeStruct((2 * NPAD, row_w), _f32),
        mesh=mesh,
        compiler_params=pltpu.CompilerParams(use_tc_tiling_on_sc=False),
        scratch_types=[
            pltpu.MemorySpace.VMEM_SHARED((NPAD, row_w), _f32),
            pltpu.VMEM((G, K), _i32),
            pltpu.VMEM((G, K), _i32),
            pltpu.VMEM((K, chw), _f32),
            pltpu.VMEM((K, chw), _f32),
            pltpu.VMEM((K, 16), _f32),
            pltpu.VMEM((K, 16), _f32),
            pltpu.VMEM((K, 16), _f32),
            pltpu.VMEM((K, 16), _f32),
            pltpu.VMEM((K, row_w), _f32),
            pltpu.VMEM((K, row_w), _f32),
            pltpu.VMEM((LANES,), _f32),
            pltpu.VMEM((RCH, row_w), _f32),
            dma, dma, dma, dma, dma, dma, dma, dma,
        ],
    )


_edge1 = _make_edge_kernel(ROW1, D1 // LANES, HEADS)
_edge2 = _make_edge_kernel(ROW2, CH2 // LANES, 1)


def _head_selector(att, heads, hid):
    a = (att.reshape(heads, hid, 1) *
         jnp.eye(heads, dtype=_f32)[:, None, :]).reshape(heads * hid, heads)
    return jnp.pad(a, ((0, 0), (0, 16 - heads)))


def kernel(x, edge_index, W1, att_src1, att_dst1, b1, W2, att_src2,
           att_dst2, b2):
    src = edge_index[0].astype(_i32).reshape(NW * NCHUNK, K)
    dst = edge_index[1].astype(_i32).reshape(NW * NCHUNK, K)

    idx = jnp.arange(D1)
    perm = (idx % HEADS) * HID + idx // HEADS
    w1p = W1[:, perm]
    a1s = _head_selector(att_src1, HEADS, HID)[perm, :]
    a1d = _head_selector(att_dst1, HEADS, HID)[perm, :]
    w2p = jnp.pad(W2[perm, :], ((0, 0), (0, CH2 - DOUT)))
    a2s = jnp.pad(att_src2.reshape(DOUT, 1), ((0, CH2 - DOUT), (0, 15)))
    a2d = jnp.pad(att_dst2.reshape(DOUT, 1), ((0, CH2 - DOUT), (0, 15)))
    rsel = (jnp.arange(D1)[None, :] // HID ==
            jnp.arange(16)[:, None]).astype(_f32)[:, perm]
    b1r = b1[perm].reshape(1, D1)
    b2r = b2.reshape(1, DOUT)

    h1, ap1, bp1, g1 = _dense1(x, w1p, a1s, a1d)
    acc1 = _edge1(h1, ap1, bp1, g1, src, dst)
    h2, ap2, bp2, g2 = _dense2(acc1, b1r, w2p, a2s, a2d, rsel)
    acc2 = _edge2(h2, ap2, bp2, g2, src, dst)
    return _dense3(acc2, b2r)


# speedup vs baseline: 85.7264x; 1.6976x over previous
import functools

import jax
import jax.numpy as jnp
from jax import lax
from jax.experimental import pallas as pl
from jax.experimental.pallas import tpu as pltpu
from jax.experimental.pallas import tpu_sc as plsc

N = 10000
E = 320000
DIN = 128
HEADS = 8
HID = 16
D1 = HEADS * HID
DOUT = 40

NC = 2
NS = 16
LANES = 16
NW = NC * NS
EPT = E // NW
K1 = 40
K2 = 80
G = 25
NPAD = 10240
RPT = NPAD // NS
RCH = 64

ROW1 = D1 + 16
CH2 = 48
ROW2 = CH2 + 16

_f32 = jnp.float32
_i32 = jnp.int32


def _dense1_body(x_ref, w1_ref, as_ref, ad_ref, he_ref, bp_ref, g_ref):
    h = jnp.dot(x_ref[...], w1_ref[...], preferred_element_type=_f32)
    ap = jnp.dot(h, as_ref[...], preferred_element_type=_f32)
    bp = jnp.dot(h, ad_ref[...], preferred_element_type=_f32)
    he_ref[:, 0:D1] = h
    he_ref[:, D1:ROW1] = ap
    bp_ref[...] = bp
    gm = jnp.max(ap, axis=0) + jnp.max(bp, axis=0)
    g = jnp.maximum(gm, 0.2 * gm)
    g_ref[...] = jnp.broadcast_to(g[None, :], (8, 16))


_dense1 = pl.pallas_call(
    _dense1_body,
    out_shape=[
        jax.ShapeDtypeStruct((N, ROW1), _f32),
        jax.ShapeDtypeStruct((N, 16), _f32),
        jax.ShapeDtypeStruct((8, 16), _f32),
    ],
)


def _dense2_body(acc_ref, b1_ref, w2p_ref, a2s_ref, a2d_ref, r_ref,
                 he_ref, bp2_ref, g2_ref):
    num = acc_ref[0:N, 0:D1] + acc_ref[NPAD:NPAD + N, 0:D1]
    den8 = acc_ref[0:N, D1:ROW1] + acc_ref[NPAD:NPAD + N, D1:ROW1]
    den128 = jnp.dot(den8, r_ref[...], preferred_element_type=_f32)
    z = jnp.maximum(num / (den128 + 1e-16) + b1_ref[...], 0.0)
    h2 = jnp.dot(z, w2p_ref[...], preferred_element_type=_f32)
    ap2 = jnp.dot(h2, a2s_ref[...], preferred_element_type=_f32)
    bp2 = jnp.dot(h2, a2d_ref[...], preferred_element_type=_f32)
    he_ref[:, 0:CH2] = h2
    he_ref[:, CH2:ROW2] = ap2
    bp2_ref[...] = bp2
    gm = jnp.max(ap2, axis=0) + jnp.max(bp2, axis=0)
    g2 = jnp.maximum(gm, 0.2 * gm)
    g2_ref[...] = jnp.broadcast_to(g2[None, :], (8, 16))


_dense2 = pl.pallas_call(
    _dense2_body,
    out_shape=[
        jax.ShapeDtypeStruct((N, ROW2), _f32),
        jax.ShapeDtypeStruct((N, 16), _f32),
        jax.ShapeDtypeStruct((8, 16), _f32),
    ],
)


def _dense3_body(acc_ref, b2_ref, out_ref):
    num = acc_ref[0:N, 0:DOUT] + acc_ref[NPAD:NPAD + N, 0:DOUT]
    den = (acc_ref[0:N, CH2:CH2 + 1]
           + acc_ref[NPAD:NPAD + N, CH2:CH2 + 1])
    o = num / (den + 1e-16) + b2_ref[...]
    m = jnp.max(o, axis=1, keepdims=True)
    e = jnp.exp(o - m)
    out_ref[...] = o - m - jnp.log(jnp.sum(e, axis=1, keepdims=True))


_dense3 = pl.pallas_call(
    _dense3_body,
    out_shape=jax.ShapeDtypeStruct((N, DOUT), _f32),
)


def _edge_body(kk, ngroup, row_w, n_hvec, heads, h_hbm, bp_hbm, g_hbm,
               src_hbm, dst_hbm, out_hbm, acc, srcg, dstg, rows0, rows1,
               bv0, bv1, out0, out1, gv, zbuf,
               sr0, sr1, sb0, sb1, sc0, sc1):
    c = lax.axis_index("c")
    s = lax.axis_index("s")
    wid = s * NC + c
    chw = n_hvec * LANES
    rows_ = (rows0, rows1)
    bv_ = (bv0, bv1)
    out_ = (out0, out1)
    scsem = (sc0, sc1)
    sems = ((sr0, sb0), (sr1, sb1))
    nchunk = EPT // kk

    pltpu.sync_copy(g_hbm.at[0], gv)

    def zrow(r, carry):
        for j in range(row_w // LANES):
            zbuf[r, pl.ds(j * LANES, LANES)] = jnp.zeros((LANES,), _f32)
        return carry

    lax.fori_loop(0, RCH, zrow, 0)
    for j in range(RPT // RCH):
        pltpu.sync_copy(zbuf, acc.at[pl.ds(s * RPT + j * RCH, RCH)])
    plsc.subcore_barrier()

    gvec = gv[...]
    widx = (lax.iota(_i32, LANES) % heads)[:, None]
    dnums = lax.GatherDimensionNumbers(
        offset_dims=(), collapsed_slice_dims=(0,), start_index_map=(0,))
    wbase = wid * nchunk

    def fire(j, b):
        d0 = pltpu.async_copy(h_hbm.at[srcg.at[j]], rows_[b], sems[b][0])
        d1 = pltpu.async_copy(bp_hbm.at[dstg.at[j]], bv_[b], sems[b][1])
        return (d0, d1)

    def compute(b):
        rv, bvb, ov = rows_[b], bv_[b], out_[b]

        def edge2(i, icarry):
            for u in range(2):
                k = i * 2 + u
                t = rv[k, pl.ds(chw, LANES)] + bvb[k, :]
                w = jnp.exp(jnp.maximum(t, 0.2 * t) - gvec)
                ov[k, pl.ds(chw, LANES)] = w
                wp = lax.gather(
                    w, widx, dnums, (1,),
                    mode=lax.GatherScatterMode.PROMISE_IN_BOUNDS)
                for h in range(n_hvec):
                    ov[k, pl.ds(h * LANES, LANES)] = (
                        wp * rv[k, pl.ds(h * LANES, LANES)])
            return icarry

        lax.fori_loop(0, kk // 2, edge2, 0)

    def group(g, carry):
        pltpu.sync_copy(src_hbm.at[pl.ds(wbase + g * G, G)], srcg)
        pltpu.sync_copy(dst_hbm.at[pl.ds(wbase + g * G, G)], dstg)
        pend = fire(0, 0)
        pend_sc = [None, None]
        for j in range(G):
            b = j % 2
            nxt = fire(j + 1, 1 - b) if j + 1 < G else None
            for d in pend:
                d.wait()
            if pend_sc[b] is not None:
                pend_sc[b].wait()
            compute(b)
            pend_sc[b] = pltpu.async_copy(
                out_[b], acc.at[dstg.at[j]], scsem[b], add=True)
            pend = nxt
        for d in pend_sc:
            if d is not None:
                d.wait()
        return carry

    lax.fori_loop(0, ngroup, group, 0)
    plsc.subcore_barrier()

    for j in range(RPT // RCH):
        start = s * RPT + j * RCH
        pltpu.sync_copy(acc.at[pl.ds(start, RCH)], zbuf)
        pltpu.sync_copy(zbuf, out_hbm.at[pl.ds(c * NPAD + start, RCH)])


def _make_edge_kernel(kk, row_w, n_hvec, heads):
    mesh = plsc.VectorSubcoreMesh(
        core_axis_name="c", subcore_axis_name="s",
        num_cores=NC, num_subcores=NS)
    ngroup = EPT // kk // G
    dma = pltpu.SemaphoreType.DMA
    return pl.kernel(
        functools.partial(_edge_body, kk, ngroup, row_w, n_hvec, heads),
        out_type=jax.ShapeDtypeStruct((2 * NPAD, row_w), _f32),
        mesh=mesh,
        compiler_params=pltpu.CompilerParams(use_tc_tiling_on_sc=False),
        scratch_types=[
            pltpu.MemorySpace.VMEM_SHARED((NPAD, row_w), _f32),
            pltpu.VMEM((G, kk), _i32),
            pltpu.VMEM((G, kk), _i32),
            pltpu.VMEM((kk, row_w), _f32),
            pltpu.VMEM((kk, row_w), _f32),
            pltpu.VMEM((kk, 16), _f32),
            pltpu.VMEM((kk, 16), _f32),
            pltpu.VMEM((kk, row_w), _f32),
            pltpu.VMEM((kk, row_w), _f32),
            pltpu.VMEM((LANES,), _f32),
            pltpu.VMEM((RCH, row_w), _f32),
            dma, dma, dma, dma, dma, dma,
        ],
    )


_edge1 = _make_edge_kernel(K1, ROW1, D1 // LANES, HEADS)
_edge2 = _make_edge_kernel(K2, ROW2, CH2 // LANES, 1)


def _head_selector(att, heads, hid):
    a = (att.reshape(heads, hid, 1) *
         jnp.eye(heads, dtype=_f32)[:, None, :]).reshape(heads * hid, heads)
    return jnp.pad(a, ((0, 0), (0, 16 - heads)))


def kernel(x, edge_index, W1, att_src1, att_dst1, b1, W2, att_src2,
           att_dst2, b2):
    src1 = edge_index[0].astype(_i32).reshape(E // K1, K1)
    dst1 = edge_index[1].astype(_i32).reshape(E // K1, K1)
    src2 = edge_index[0].astype(_i32).reshape(E // K2, K2)
    dst2 = edge_index[1].astype(_i32).reshape(E // K2, K2)

    idx = jnp.arange(D1)
    perm = (idx % HEADS) * HID + idx // HEADS
    w1p = W1[:, perm]
    a1s = _head_selector(att_src1, HEADS, HID)[perm, :]
    a1d = _head_selector(att_dst1, HEADS, HID)[perm, :]
    w2p = jnp.pad(W2[perm, :], ((0, 0), (0, CH2 - DOUT)))
    a2s = jnp.pad(att_src2.reshape(DOUT, 1), ((0, CH2 - DOUT), (0, 15)))
    a2d = jnp.pad(att_dst2.reshape(DOUT, 1), ((0, CH2 - DOUT), (0, 15)))
    rsel = (jnp.arange(D1)[None, :] // HID ==
            jnp.arange(16)[:, None]).astype(_f32)[:, perm]
    b1r = b1[perm].reshape(1, D1)
    b2r = b2.reshape(1, DOUT)

    h1e, bp1, g1 = _dense1(x, w1p, a1s, a1d)
    acc1 = _edge1(h1e, bp1, g1, src1, dst1)
    h2e, bp2, g2 = _dense2(acc1, b1r, w2p, a2s, a2d, rsel)
    acc2 = _edge2(h2e, bp2, g2, src2, dst2)
    return _dense3(acc2, b2r)


# speedup vs baseline: 86.2333x; 1.0059x over previous
import functools

import jax
import jax.numpy as jnp
from jax import lax
from jax.experimental import pallas as pl
from jax.experimental.pallas import tpu as pltpu
from jax.experimental.pallas import tpu_sc as plsc

N = 10000
E = 320000
DIN = 128
HEADS = 8
HID = 16
D1 = HEADS * HID
DOUT = 40

NC = 2
NS = 16
LANES = 16
NW = NC * NS
EPT = E // NW
K1 = 40
K2 = 80
G = 25
NPAD = 10240
RPT = NPAD // NS
RCH = 64

ROW1 = D1 + 16
CH2 = 48
ROW2 = CH2 + 16

_f32 = jnp.float32
_i32 = jnp.int32


def _dense1_body(x_ref, w1_ref, as_ref, ad_ref, he_ref, bp_ref, g_ref):
    h = jnp.dot(x_ref[...], w1_ref[...], preferred_element_type=_f32)
    ap = jnp.dot(h, as_ref[...], preferred_element_type=_f32)
    bp = jnp.dot(h, ad_ref[...], preferred_element_type=_f32)
    he_ref[:, 0:D1] = h
    he_ref[:, D1:ROW1] = ap
    bp_ref[...] = bp
    gm = jnp.max(ap, axis=0) + jnp.max(bp, axis=0)
    g = jnp.maximum(gm, 0.2 * gm)
    g_ref[...] = jnp.broadcast_to(g[None, :], (8, 16))


_dense1 = pl.pallas_call(
    _dense1_body,
    out_shape=[
        jax.ShapeDtypeStruct((N, ROW1), _f32),
        jax.ShapeDtypeStruct((N, 16), _f32),
        jax.ShapeDtypeStruct((8, 16), _f32),
    ],
)


def _dense2_body(acc_ref, b1_ref, w2p_ref, a2s_ref, a2d_ref, r_ref,
                 he_ref, bp2_ref, g2_ref):
    num = acc_ref[0:N, 0:D1] + acc_ref[NPAD:NPAD + N, 0:D1]
    den8 = acc_ref[0:N, D1:ROW1] + acc_ref[NPAD:NPAD + N, D1:ROW1]
    den128 = jnp.dot(den8, r_ref[...], preferred_element_type=_f32)
    z = jnp.maximum(num / (den128 + 1e-16) + b1_ref[...], 0.0)
    h2 = jnp.dot(z, w2p_ref[...], preferred_element_type=_f32)
    ap2 = jnp.dot(h2, a2s_ref[...], preferred_element_type=_f32)
    bp2 = jnp.dot(h2, a2d_ref[...], preferred_element_type=_f32)
    he_ref[:, 0:CH2] = h2
    he_ref[:, CH2:ROW2] = ap2
    bp2_ref[...] = bp2
    gm = jnp.max(ap2, axis=0) + jnp.max(bp2, axis=0)
    g2 = jnp.maximum(gm, 0.2 * gm)
    g2_ref[...] = jnp.broadcast_to(g2[None, :], (8, 16))


_dense2 = pl.pallas_call(
    _dense2_body,
    out_shape=[
        jax.ShapeDtypeStruct((N, ROW2), _f32),
        jax.ShapeDtypeStruct((N, 16), _f32),
        jax.ShapeDtypeStruct((8, 16), _f32),
    ],
)


def _dense3_body(acc_ref, b2_ref, out_ref):
    num = acc_ref[0:N, 0:DOUT] + acc_ref[NPAD:NPAD + N, 0:DOUT]
    den = (acc_ref[0:N, CH2:CH2 + 1]
           + acc_ref[NPAD:NPAD + N, CH2:CH2 + 1])
    o = num / (den + 1e-16) + b2_ref[...]
    m = jnp.max(o, axis=1, keepdims=True)
    e = jnp.exp(o - m)
    out_ref[...] = o - m - jnp.log(jnp.sum(e, axis=1, keepdims=True))


_dense3 = pl.pallas_call(
    _dense3_body,
    out_shape=jax.ShapeDtypeStruct((N, DOUT), _f32),
)


def _edge_body(kk, ngroup, row_w, n_hvec, heads, h_hbm, bp_hbm, g_hbm,
               src_hbm, dst_hbm, out_hbm, acc, srcg, dstg, rows0, rows1,
               bv0, bv1, out0, out1, gv, zbuf,
               sr0, sr1, sb0, sb1, sc0, sc1):
    c = lax.axis_index("c")
    s = lax.axis_index("s")
    wid = s * NC + c
    chw = n_hvec * LANES
    rows_ = (rows0, rows1)
    bv_ = (bv0, bv1)
    out_ = (out0, out1)
    scsem = (sc0, sc1)
    sems = ((sr0, sb0), (sr1, sb1))
    nchunk = EPT // kk

    pltpu.sync_copy(g_hbm.at[0], gv)

    def zrow(r, carry):
        for j in range(row_w // LANES):
            zbuf[r, pl.ds(j * LANES, LANES)] = jnp.zeros((LANES,), _f32)
        return carry

    lax.fori_loop(0, RCH, zrow, 0)
    zd = [pltpu.async_copy(zbuf, acc.at[pl.ds(s * RPT + j * RCH, RCH)], sr0)
          for j in range(RPT // RCH)]
    for d in zd:
        d.wait()
    plsc.subcore_barrier()

    gvec = gv[...]
    widx = (lax.iota(_i32, LANES) % heads)[:, None]
    dnums = lax.GatherDimensionNumbers(
        offset_dims=(), collapsed_slice_dims=(0,), start_index_map=(0,))
    wbase = wid * nchunk

    def fire(j, b):
        d0 = pltpu.async_copy(h_hbm.at[srcg.at[j]], rows_[b], sems[b][0])
        d1 = pltpu.async_copy(bp_hbm.at[dstg.at[j]], bv_[b], sems[b][1])
        return (d0, d1)

    def compute(b):
        rv, bvb, ov = rows_[b], bv_[b], out_[b]

        def edge2(i, icarry):
            for u in range(2):
                k = i * 2 + u
                t = rv[k, pl.ds(chw, LANES)] + bvb[k, :]
                w = jnp.exp(jnp.maximum(t, 0.2 * t) - gvec)
                ov[k, pl.ds(chw, LANES)] = w
                wp = lax.gather(
                    w, widx, dnums, (1,),
                    mode=lax.GatherScatterMode.PROMISE_IN_BOUNDS)
                for h in range(n_hvec):
                    ov[k, pl.ds(h * LANES, LANES)] = (
                        wp * rv[k, pl.ds(h * LANES, LANES)])
            return icarry

        lax.fori_loop(0, kk // 2, edge2, 0)

    def group(g, carry):
        pltpu.sync_copy(src_hbm.at[pl.ds(wbase + g * G, G)], srcg)
        pltpu.sync_copy(dst_hbm.at[pl.ds(wbase + g * G, G)], dstg)
        pend = fire(0, 0)
        pend_sc = [None, None]
        for j in range(G):
            b = j % 2
            nxt = fire(j + 1, 1 - b) if j + 1 < G else None
            for d in pend:
                d.wait()
            if pend_sc[b] is not None:
                pend_sc[b].wait()
            compute(b)
            pend_sc[b] = pltpu.async_copy(
                out_[b], acc.at[dstg.at[j]], scsem[b], add=True)
            pend = nxt
        for d in pend_sc:
            if d is not None:
                d.wait()
        return carry

    lax.fori_loop(0, ngroup, group, 0)
    plsc.subcore_barrier()

    rd = [pltpu.async_copy(
        acc.at[pl.ds(s * RPT + j * RCH, RCH)],
        out_hbm.at[pl.ds(c * NPAD + s * RPT + j * RCH, RCH)], sr1)
        for j in range(RPT // RCH)]
    for d in rd:
        d.wait()


def _make_edge_kernel(kk, row_w, n_hvec, heads):
    mesh = plsc.VectorSubcoreMesh(
        core_axis_name="c", subcore_axis_name="s",
        num_cores=NC, num_subcores=NS)
    ngroup = EPT // kk // G
    dma = pltpu.SemaphoreType.DMA
    return pl.kernel(
        functools.partial(_edge_body, kk, ngroup, row_w, n_hvec, heads),
        out_type=jax.ShapeDtypeStruct((2 * NPAD, row_w), _f32),
        mesh=mesh,
        compiler_params=pltpu.CompilerParams(use_tc_tiling_on_sc=False),
        scratch_types=[
            pltpu.MemorySpace.VMEM_SHARED((NPAD, row_w), _f32),
            pltpu.VMEM((G, kk), _i32),
            pltpu.VMEM((G, kk), _i32),
            pltpu.VMEM((kk, row_w), _f32),
            pltpu.VMEM((kk, row_w), _f32),
            pltpu.VMEM((kk, 16), _f32),
            pltpu.VMEM((kk, 16), _f32),
            pltpu.VMEM((kk, row_w), _f32),
            pltpu.VMEM((kk, row_w), _f32),
            pltpu.VMEM((LANES,), _f32),
            pltpu.VMEM((RCH, row_w), _f32),
            dma, dma, dma, dma, dma, dma,
        ],
    )


_edge1 = _make_edge_kernel(K1, ROW1, D1 // LANES, HEADS)
_edge2 = _make_edge_kernel(K2, ROW2, CH2 // LANES, 1)


def _head_selector(att, heads, hid):
    a = (att.reshape(heads, hid, 1) *
         jnp.eye(heads, dtype=_f32)[:, None, :]).reshape(heads * hid, heads)
    return jnp.pad(a, ((0, 0), (0, 16 - heads)))


def kernel(x, edge_index, W1, att_src1, att_dst1, b1, W2, att_src2,
           att_dst2, b2):
    src1 = edge_index[0].astype(_i32).reshape(E // K1, K1)
    dst1 = edge_index[1].astype(_i32).reshape(E // K1, K1)
    src2 = edge_index[0].astype(_i32).reshape(E // K2, K2)
    dst2 = edge_index[1].astype(_i32).reshape(E // K2, K2)

    idx = jnp.arange(D1)
    perm = (idx % HEADS) * HID + idx // HEADS
    w1p = W1[:, perm]
    a1s = _head_selector(att_src1, HEADS, HID)[perm, :]
    a1d = _head_selector(att_dst1, HEADS, HID)[perm, :]
    w2p = jnp.pad(W2[perm, :], ((0, 0), (0, CH2 - DOUT)))
    a2s = jnp.pad(att_src2.reshape(DOUT, 1), ((0, CH2 - DOUT), (0, 15)))
    a2d = jnp.pad(att_dst2.reshape(DOUT, 1), ((0, CH2 - DOUT), (0, 15)))
    rsel = (jnp.arange(D1)[None, :] // HID ==
            jnp.arange(16)[:, None]).astype(_f32)[:, perm]
    b1r = b1[perm].reshape(1, D1)
    b2r = b2.reshape(1, DOUT)

    h1e, bp1, g1 = _dense1(x, w1p, a1s, a1d)
    acc1 = _edge1(h1e, bp1, g1, src1, dst1)
    h2e, bp2, g2 = _dense2(acc1, b1r, w2p, a2s, a2d, rsel)
    acc2 = _edge2(h2e, bp2, g2, src2, dst2)
    return _dense3(acc2, b2r)
